# Initial kernel scaffold; baseline (speedup 1.0000x reference)
import jax
import jax.numpy as jnp
from jax.experimental import pallas as pl


def kernel(img1, img2, y, W_cls, W_feat, prototypes_t, com_prototypes):
    raise NotImplementedError("write your pallas kernel here")


# speedup vs baseline: 143.5531x; 143.5531x over previous
import math

import jax
import jax.numpy as jnp
from jax.experimental import pallas as pl

B = 1024
D = 3 * 32 * 32
NC = 10
NCP = 16
FEAT = 128
W_EMA = 0.99
LOG_W = math.log(W_EMA)


def _pcl_kernel(x1_ref, x2_ref, wc_ref, wf_ref, y_ref, pt0_ref, cp0_ref,
                out1_ref, out2_ref, lp_ref, lp2_ref, plab_ref, lc_ref,
                feat1_ref):
    x1 = x1_ref[...]
    x2 = x2_ref[...]
    wc = wc_ref[...]
    wf = wf_ref[...]

    out1 = jnp.dot(x1, wc, preferred_element_type=jnp.float32)
    out2 = jnp.dot(x2, wc, preferred_element_type=jnp.float32)
    f1 = jnp.dot(x1, wf, preferred_element_type=jnp.float32)
    f2 = jnp.dot(x2, wf, preferred_element_type=jnp.float32)

    def l2n(v):
        n = jnp.sqrt(jnp.sum(v * v, axis=1, keepdims=True))
        return v / jnp.maximum(n, 1e-12)

    f1 = l2n(f1)
    f2 = l2n(f2)
    out1_ref[...] = out1
    out2_ref[...] = out2
    feat1_ref[...] = f1

    col = jax.lax.broadcasted_iota(jnp.int32, (B, NCP), 1)
    valid = col < NC
    masked = jnp.where(valid, out1, -jnp.inf)
    mx = jnp.max(masked, axis=1, keepdims=True)
    idx = jnp.min(jnp.where(masked == mx, col, NCP), axis=1, keepdims=True)
    plab_ref[...] = idx

    def ema(labels_col, p0):
        onehot = (labels_col == col).astype(jnp.float32)
        csum = onehot
        d = 1
        while d < B:
            shifted = jnp.concatenate(
                [jnp.zeros((d, NCP), jnp.float32), csum[:B - d, :]], axis=0)
            csum = csum + shifted
            d *= 2
        krow = csum[B - 1:B, :]
        j = jnp.sum(onehot * csum, axis=1, keepdims=True)
        k_i = jnp.sum(onehot * krow, axis=1, keepdims=True)
        r = k_i - j
        wts = (1.0 - W_EMA) * jnp.exp(r * LOG_W)
        A = onehot * wts
        psum = jax.lax.dot_general(A, f1, (((0,), (0,)), ((), ())),
                                   preferred_element_type=jnp.float32)
        ones = jnp.ones((B, 1), jnp.float32)
        colsum = jax.lax.dot_general(A, ones, (((0,), (0,)), ((), ())),
                                     preferred_element_type=jnp.float32)
        p = psum + (1.0 - colsum) * p0
        return l2n(p)

    pt = ema(idx, pt0_ref[...])
    cp = ema(y_ref[...], cp0_ref[...])

    dn = (((1,), (1,)), ((), ()))
    lp_ref[...] = jax.lax.dot_general(f2, pt, dn,
                                      preferred_element_type=jnp.float32)
    lp2_ref[...] = jax.lax.dot_general(f1, pt, dn,
                                       preferred_element_type=jnp.float32)
    lc_ref[...] = jax.lax.dot_general(f2, cp, dn,
                                      preferred_element_type=jnp.float32)


def kernel(img1, img2, y, W_cls, W_feat, prototypes_t, com_prototypes):
    x1 = img1.reshape(B, D)
    x2 = img2.reshape(B, D)
    wc = jnp.pad(W_cls, ((0, 0), (0, NCP - NC)))
    y2 = y.astype(jnp.int32).reshape(B, 1)
    pt0 = jnp.pad(prototypes_t, ((0, NCP - NC), (0, 0)))
    cp0 = jnp.pad(com_prototypes, ((0, NCP - NC), (0, 0)))

    f32 = jnp.float32
    out_shape = (
        jax.ShapeDtypeStruct((B, NCP), f32),
        jax.ShapeDtypeStruct((B, NCP), f32),
        jax.ShapeDtypeStruct((B, NCP), f32),
        jax.ShapeDtypeStruct((B, NCP), f32),
        jax.ShapeDtypeStruct((B, 1), jnp.int32),
        jax.ShapeDtypeStruct((B, NCP), f32),
        jax.ShapeDtypeStruct((B, FEAT), f32),
    )
    out1p, out2p, lpp, lp2p, plab, lcp, feat1 = pl.pallas_call(
        _pcl_kernel, out_shape=out_shape,
    )(x1, x2, wc, W_feat, y2, pt0, cp0)

    return (out1p[:, :NC], out2p[:, :NC], lpp[:, :NC], lp2p[:, :NC],
            plab.reshape(B), lcp[:, :NC], feat1)
